# baseline (speedup 1.0000x reference)
import functools

import jax
import jax.numpy as jnp
from jax import lax
from jax.experimental import pallas as pl
from jax.experimental.pallas import tpu as pltpu
from jax.experimental.pallas import tpu_sc as plsc

_NC, _NS, _LANES = 2, 16, 16
_NW = _NC * _NS


def kernel(tokens, pos_emb):
    B, L, D = tokens.shape
    rows_per_w = L // _NW
    C = 32
    n_chunks = rows_per_w // C
    CHUNK = C * D

    tok_flat = tokens.reshape(B * L * D)
    pe_flat = pos_emb.reshape(-1)

    mesh = plsc.VectorSubcoreMesh(core_axis_name="c", subcore_axis_name="s")

    @functools.partial(
        pl.kernel,
        out_type=jax.ShapeDtypeStruct((B * L * D,), jnp.float32),
        mesh=mesh,
        scratch_types=[
            pltpu.VMEM((CHUNK,), jnp.float32),
            pltpu.VMEM((CHUNK,), jnp.float32),
        ],
    )
    def sc_add(tok_hbm, pe_hbm, out_hbm, pe_buf, tok_buf):
        wid = lax.axis_index("s") * _NC + lax.axis_index("c")
        base = wid * rows_per_w * D
        for c in range(n_chunks):
            pe_off = base + c * CHUNK
            pltpu.sync_copy(pe_hbm.at[pl.ds(pe_off, CHUNK)], pe_buf)
            for b in range(B):
                t_off = b * L * D + pe_off
                pltpu.sync_copy(tok_hbm.at[pl.ds(t_off, CHUNK)], tok_buf)

                @plsc.parallel_loop(0, CHUNK, step=_LANES, unroll=8)
                def _(i):
                    tok_buf[pl.ds(i, _LANES)] = (
                        tok_buf[pl.ds(i, _LANES)] + pe_buf[pl.ds(i, _LANES)]
                    )

                pltpu.sync_copy(tok_buf, out_hbm.at[pl.ds(t_off, CHUNK)])

    out = sc_add(tok_flat, pe_flat)
    return out.reshape(B, L, D)


# speedup vs baseline: 1.1628x; 1.1628x over previous
import functools

import jax
import jax.numpy as jnp
from jax import lax
from jax.experimental import pallas as pl
from jax.experimental.pallas import tpu as pltpu
from jax.experimental.pallas import tpu_sc as plsc

_NC, _NS, _LANES = 2, 16, 16
_NW = _NC * _NS


def kernel(tokens, pos_emb):
    B, L, D = tokens.shape
    rows_per_w = L // _NW
    QR = 32
    C = 16
    n_q = rows_per_w // QR
    cpq = QR // C
    CHUNK = C * D
    items = []
    for q in range(n_q):
        for b in range(B):
            for c in range(cpq):
                items.append((q, b, c))
    n_items = len(items)

    tok_flat = tokens.reshape(B * L * D)
    pe_flat = pos_emb.reshape(-1)

    mesh = plsc.VectorSubcoreMesh(core_axis_name="c", subcore_axis_name="s")

    @functools.partial(
        pl.kernel,
        out_type=jax.ShapeDtypeStruct((B * L * D,), jnp.float32),
        mesh=mesh,
        scratch_types=[
            pltpu.VMEM((QR * D,), jnp.float32),
            [pltpu.VMEM((CHUNK,), jnp.float32) for _ in range(2)],
            [pltpu.VMEM((CHUNK,), jnp.float32) for _ in range(2)],
            [pltpu.SemaphoreType.DMA for _ in range(4)],
        ],
    )
    def sc_add(tok_hbm, pe_hbm, out_hbm, pe_buf, in_bufs, out_bufs, sems):
        wid = lax.axis_index("s") * _NC + lax.axis_index("c")
        base = wid * rows_per_w * D

        def tok_off(k):
            q, b, c = items[k]
            return b * L * D + base + (q * QR + c * C) * D

        in_d = [None] * n_items
        out_d = [None] * n_items
        for k in range(2):
            in_d[k] = pltpu.async_copy(
                tok_hbm.at[pl.ds(tok_off(k), CHUNK)], in_bufs[k % 2], sems[k % 2]
            )
        for k in range(n_items):
            p = k % 2
            q, b, c = items[k]
            if b == 0 and c == 0:
                pltpu.sync_copy(
                    pe_hbm.at[pl.ds(base + q * QR * D, QR * D)], pe_buf
                )
            in_d[k].wait()
            if k >= 2:
                out_d[k - 2].wait()
            pe_base = c * CHUNK

            @plsc.parallel_loop(0, CHUNK, step=_LANES, unroll=8)
            def _(i):
                out_bufs[p][pl.ds(i, _LANES)] = (
                    in_bufs[p][pl.ds(i, _LANES)] + pe_buf[pl.ds(pe_base + i, _LANES)]
                )

            out_d[k] = pltpu.async_copy(
                out_bufs[p], out_hbm.at[pl.ds(tok_off(k), CHUNK)], sems[2 + p]
            )
            if k + 2 < n_items:
                in_d[k + 2] = pltpu.async_copy(
                    tok_hbm.at[pl.ds(tok_off(k + 2), CHUNK)], in_bufs[p], sems[p]
                )
        out_d[n_items - 2].wait()
        out_d[n_items - 1].wait()

    out = sc_add(tok_flat, pe_flat)
    return out.reshape(B, L, D)


# speedup vs baseline: 1.1943x; 1.0271x over previous
import functools

import jax
import jax.numpy as jnp
from jax import lax
from jax.experimental import pallas as pl
from jax.experimental.pallas import tpu as pltpu
from jax.experimental.pallas import tpu_sc as plsc

_NC, _NS, _LANES = 2, 16, 16
_NW = _NC * _NS


def kernel(tokens, pos_emb):
    B, L, D = tokens.shape
    rows_per_w = L // _NW
    QR = 32
    C = 16
    n_q = rows_per_w // QR
    cpq = QR // C
    CHUNK = C * D
    items = []
    for q in range(n_q):
        for b in range(B):
            for c in range(cpq):
                items.append((q, b, c))
    n_items = len(items)
    NBUF = 4

    tok_flat = tokens.reshape(B * L * D)
    pe_flat = pos_emb.reshape(-1)

    mesh = plsc.VectorSubcoreMesh(core_axis_name="c", subcore_axis_name="s")

    @functools.partial(
        pl.kernel,
        out_type=jax.ShapeDtypeStruct((B * L * D,), jnp.float32),
        mesh=mesh,
        scratch_types=[
            pltpu.VMEM((QR * D,), jnp.float32),
            [pltpu.VMEM((CHUNK,), jnp.float32) for _ in range(NBUF)],
            [pltpu.SemaphoreType.DMA for _ in range(2 * NBUF)],
        ],
    )
    def sc_add(tok_hbm, pe_hbm, out_hbm, pe_buf, bufs, sems):
        wid = lax.axis_index("s") * _NC + lax.axis_index("c")
        base = wid * rows_per_w * D

        def tok_off(k):
            q, b, c = items[k]
            return b * L * D + base + (q * QR + c * C) * D

        in_d = [None] * (n_items + NBUF)
        out_d = [None] * n_items
        for k in range(NBUF):
            in_d[k] = pltpu.async_copy(
                tok_hbm.at[pl.ds(tok_off(k), CHUNK)], bufs[k % NBUF], sems[k % NBUF]
            )
        for k in range(n_items):
            p = k % NBUF
            q, b, c = items[k]
            if b == 0 and c == 0:
                pltpu.sync_copy(
                    pe_hbm.at[pl.ds(base + q * QR * D, QR * D)], pe_buf
                )
            if k >= 2:
                out_d[k - 2].wait()
                if k + 2 < n_items:
                    in_d[k + 2] = pltpu.async_copy(
                        tok_hbm.at[pl.ds(tok_off(k + 2), CHUNK)],
                        bufs[(k + 2) % NBUF],
                        sems[(k + 2) % NBUF],
                    )
            in_d[k].wait()
            pe_base = c * CHUNK

            @plsc.parallel_loop(0, CHUNK, step=_LANES, unroll=8)
            def _(i):
                plsc.addupdate(
                    bufs[p].at[pl.ds(i, _LANES)],
                    pe_buf[pl.ds(pe_base + i, _LANES)],
                )

            out_d[k] = pltpu.async_copy(
                bufs[p], out_hbm.at[pl.ds(tok_off(k), CHUNK)], sems[NBUF + p]
            )
        out_d[n_items - 2].wait()
        out_d[n_items - 1].wait()

    out = sc_add(tok_flat, pe_flat)
    return out.reshape(B, L, D)


# speedup vs baseline: 3.4255x; 2.8682x over previous
import functools

import jax
import jax.numpy as jnp
from jax import lax
from jax.experimental import pallas as pl
from jax.experimental.pallas import tpu as pltpu
from jax.experimental.pallas import tpu_sc as plsc

_NC, _NS, _LANES = 2, 16, 16
_NW = _NC * _NS


def kernel(tokens, pos_emb):
    B, L, D = tokens.shape
    rows_per_w = L // _NW
    QR = 32
    C = 16
    n_q = rows_per_w // QR
    cpq = QR // C
    items = []
    for q in range(n_q):
        for b in range(B):
            for c in range(cpq):
                items.append((q, b, c))
    n_items = len(items)
    NBUF = 4

    tok2d = tokens.reshape(B * L, D)

    mesh = plsc.VectorSubcoreMesh(core_axis_name="c", subcore_axis_name="s")

    @functools.partial(
        pl.kernel,
        out_type=jax.ShapeDtypeStruct((B * L, D), jnp.float32),
        mesh=mesh,
        scratch_types=[
            pltpu.VMEM((QR, D), jnp.float32),
            [pltpu.VMEM((C, D), jnp.float32) for _ in range(NBUF)],
            [pltpu.SemaphoreType.DMA for _ in range(2 * NBUF)],
        ],
    )
    def sc_add(tok_hbm, pe_hbm, out_hbm, pe_buf, bufs, sems):
        wid = lax.axis_index("s") * _NC + lax.axis_index("c")
        base = wid * rows_per_w

        def tok_row(k):
            q, b, c = items[k]
            return b * L + base + (q * QR + c * C)

        in_d = [None] * (n_items + NBUF)
        out_d = [None] * n_items
        for k in range(NBUF):
            in_d[k] = pltpu.async_copy(
                tok_hbm.at[pl.ds(tok_row(k), C)], bufs[k % NBUF], sems[k % NBUF]
            )
        for k in range(n_items):
            p = k % NBUF
            q, b, c = items[k]
            if b == 0 and c == 0:
                pltpu.sync_copy(pe_hbm.at[pl.ds(base + q * QR, QR)], pe_buf)
            if k >= 2:
                out_d[k - 2].wait()
                if k + 2 < n_items:
                    in_d[k + 2] = pltpu.async_copy(
                        tok_hbm.at[pl.ds(tok_row(k + 2), C)],
                        bufs[(k + 2) % NBUF],
                        sems[(k + 2) % NBUF],
                    )
            in_d[k].wait()
            pe_row = c * C

            @plsc.parallel_loop(0, C * D, step=_LANES, unroll=8)
            def _(i):
                r = i // D
                j = i % D
                plsc.addupdate(
                    bufs[p].at[r, pl.ds(j, _LANES)],
                    pe_buf[pe_row + r, pl.ds(j, _LANES)],
                )

            out_d[k] = pltpu.async_copy(
                bufs[p], out_hbm.at[pl.ds(tok_row(k), C)], sems[NBUF + p]
            )
        out_d[n_items - 2].wait()
        out_d[n_items - 1].wait()

    out = sc_add(tok2d, pos_emb)
    return out.reshape(B, L, D)
